# baseline (speedup 1.0000x reference)
import functools

import jax
import jax.numpy as jnp
from jax import lax
from jax.experimental import pallas as pl
from jax.experimental.pallas import tpu as pltpu
from jax.experimental.pallas import tpu_sc as plsc

_LANES = 16


def _broadcast_add_sc_rows(x2, mid, embedding, n_sc):
    _, d = x2.shape
    vecs_per_row = d // _LANES

    info = plsc.get_sparse_core_info()
    nc, ns = info.num_cores, info.num_subcores
    nw = nc * ns
    rows_per_w = n_sc // nw
    chunk_rows = 16
    half_rows = chunk_rows // 2
    n_chunks = rows_per_w // chunk_rows
    nbuf = 6

    mesh = plsc.VectorSubcoreMesh(core_axis_name="c", subcore_axis_name="s")

    @functools.partial(
        pl.kernel,
        mesh=mesh,
        out_type=jax.ShapeDtypeStruct((n_sc, d), jnp.float32),
        scratch_types=[
            pltpu.VMEM((1,), jnp.int32),
            pltpu.VMEM((1, d), jnp.float32),
            [pltpu.VMEM((chunk_rows, d), jnp.float32) for _ in range(nbuf)],
            [pltpu.SemaphoreType.DMA for _ in range(nbuf)],
            [pltpu.SemaphoreType.DMA for _ in range(nbuf)],
            [pltpu.SemaphoreType.DMA for _ in range(nbuf)],
            pltpu.SemaphoreType.DMA,
        ],
    )
    def run(x_hbm, mid_hbm, emb_hbm, out_hbm, idx_v, emb_v, bufs, isems,
            isems2, osems, gsem):
        wid = lax.axis_index("s") * nc + lax.axis_index("c")
        base = wid * rows_per_w

        def start_in(i):
            off = base + i * chunk_rows
            b = i % nbuf
            d1 = pltpu.async_copy(
                x_hbm.at[pl.ds(off, half_rows)],
                bufs[b].at[pl.ds(0, half_rows)], isems[b])
            d2 = pltpu.async_copy(
                x_hbm.at[pl.ds(off + half_rows, half_rows)],
                bufs[b].at[pl.ds(half_rows, half_rows)], isems2[b])
            return (d1, d2)

        def start_out(i):
            off = base + i * chunk_rows
            return pltpu.async_copy(
                bufs[i % nbuf], out_hbm.at[pl.ds(off, chunk_rows)],
                osems[i % nbuf])

        half = vecs_per_row // 2

        depth = nbuf - 1
        in_dma = {}
        out_dma = {}
        for i in range(min(depth, n_chunks)):
            in_dma[i] = start_in(i)

        pltpu.sync_copy(mid_hbm, idx_v)
        pltpu.async_copy(emb_hbm.at[idx_v], emb_v, gsem).wait()

        for i in range(n_chunks):
            buf = bufs[i % nbuf]
            d1, d2 = in_dma.pop(i)

            for rh, dma in ((0, d1), (1, d2)):
                dma.wait()
                r0 = rh * half_rows

                def col_body(k, carry, buf=buf, r0=r0):
                    sl = pl.ds(k * _LANES, _LANES)
                    ev = emb_v[0, sl]
                    for r in range(half_rows):
                        buf[r0 + r, sl] = buf[r0 + r, sl] + ev
                    return carry

                lax.fori_loop(0, vecs_per_row, col_body, 0)

            out_dma[i] = start_out(i)
            if i + depth < n_chunks:
                if i - 1 >= 0:
                    out_dma.pop(i - 1).wait()
                in_dma[i + depth] = start_in(i + depth)

        for i in sorted(out_dma):
            out_dma[i].wait()

    return run


def kernel(x, modality_id, embedding):
    b, t, d = x.shape
    x2 = x.reshape(b * t, d)
    mid = jnp.asarray(modality_id, jnp.int32).reshape(1)
    out2 = _broadcast_add_sc_rows(x2, mid, embedding, b * t)(
        x2, mid, embedding)
    return out2.reshape(b, t, d)


# speedup vs baseline: 1.6197x; 1.6197x over previous
import jax
import jax.numpy as jnp
from jax.experimental import pallas as pl
from jax.experimental.pallas import tpu as pltpu


def _tc_add(mid_ref, x_ref, emb_ref, o_ref):
    row = emb_ref[mid_ref[0]]
    o_ref[...] = x_ref[...] + row[None, :]


def kernel(x, modality_id, embedding):
    b, t, d = x.shape
    n = b * t
    x2 = x.reshape(n, d)
    mid = jnp.asarray(modality_id, jnp.int32).reshape(1)
    blk = 512
    grid = n // blk
    out2 = pl.pallas_call(
        _tc_add,
        grid_spec=pltpu.PrefetchScalarGridSpec(
            num_scalar_prefetch=1,
            grid=(grid,),
            in_specs=[
                pl.BlockSpec((blk, d), lambda i, mid: (i, 0)),
                pl.BlockSpec((2, d), lambda i, mid: (0, 0)),
            ],
            out_specs=pl.BlockSpec((blk, d), lambda i, mid: (i, 0)),
        ),
        out_shape=jax.ShapeDtypeStruct((n, d), jnp.float32),
    )(mid, x2, embedding)
    return out2.reshape(b, t, d)
